# Initial kernel scaffold; baseline (speedup 1.0000x reference)
import jax
import jax.numpy as jnp
from jax.experimental import pallas as pl


def kernel(z_e_x, emb):
    raise NotImplementedError("write your pallas kernel here")


# speedup vs baseline: 1.4919x; 1.4919x over previous
import jax
import jax.numpy as jnp
from jax.experimental import pallas as pl


def _vq_body(zt_ref, et_ref, o_ref):
    hw = zt_ref.shape[1]
    d_dim = zt_ref.shape[2]
    k = et_ref.shape[1]
    zt = zt_ref[0]
    et = et_ref[...]
    acc = jnp.zeros((hw, k), jnp.float32)
    for d in range(d_dim):
        zd = zt[:, d][:, None]
        ed = et[d, :][None, :]
        diff = zd - ed
        acc = acc + diff * diff
    min_val = jnp.min(acc, axis=1, keepdims=True)
    idx = jax.lax.broadcasted_iota(jnp.int32, (hw, k), 1)
    masked = jnp.where(acc == min_val, idx, k)
    o_ref[0, 0, :] = jnp.min(masked, axis=1)


def kernel(z_e_x, emb):
    b, d, h, w = z_e_x.shape
    k = emb.shape[0]
    hw = h * w
    zt = z_e_x.reshape(b, d, hw).transpose(0, 2, 1)
    et = emb.T
    out = pl.pallas_call(
        _vq_body,
        grid=(b,),
        in_specs=[
            pl.BlockSpec((1, hw, d), lambda i: (i, 0, 0)),
            pl.BlockSpec((d, k), lambda i: (0, 0)),
        ],
        out_specs=pl.BlockSpec((1, 1, hw), lambda i: (i, 0, 0)),
        out_shape=jax.ShapeDtypeStruct((b, 1, hw), jnp.int32),
    )(zt, et)
    return out.reshape(b, h, w)
